# baseline (speedup 1.0000x reference)
import jax
import jax.numpy as jnp
from jax.experimental import pallas as pl

_G = 16
_K = 200
_N = 4


def _wide_kernel(x_ref, w_ref, b_ref, o_ref):
    o_ref[...] = (
        jnp.dot(x_ref[...], w_ref[...], preferred_element_type=jnp.float32)
        + b_ref[...]
    )


def kernel(feat, server_model, len_seq, mask, manfeat, emb1_w, emb2_w, emb3_w,
           emb4_w, emb5_w, k_w, o_w, cls_w, cls_b, wide_w, wide_b):
    b, k = manfeat.shape
    bg = b // _G
    x3 = manfeat.reshape(bg, _G * _K)
    w3 = jnp.kron(jnp.eye(_G, dtype=jnp.float32), wide_w)
    b3 = jnp.tile(wide_b, _G).reshape(1, _G * _N)
    out = pl.pallas_call(
        _wide_kernel,
        grid=(1,),
        in_specs=[
            pl.BlockSpec((bg, _G * _K), lambda i: (0, 0)),
            pl.BlockSpec((_G * _K, _G * _N), lambda i: (0, 0)),
            pl.BlockSpec((1, _G * _N), lambda i: (0, 0)),
        ],
        out_specs=pl.BlockSpec((bg, _G * _N), lambda i: (0, 0)),
        out_shape=jax.ShapeDtypeStruct((bg, _G * _N), jnp.float32),
    )(x3, w3, b3)
    return out.reshape(b, _N)


# speedup vs baseline: 1.1417x; 1.1417x over previous
import jax
import jax.numpy as jnp
from jax.experimental import pallas as pl
from jax.experimental.pallas import tpu as pltpu

_CHUNKS = 16
_ROWS = 256


def _wide_kernel(x_hbm, w_ref, b_ref, o_hbm, x_vmem, o_vmem, sem_in, sem_out):
    in_cp = [
        pltpu.make_async_copy(
            x_hbm.at[pl.ds(i * _ROWS, _ROWS), :],
            x_vmem.at[pl.ds(i * _ROWS, _ROWS), :],
            sem_in.at[i],
        )
        for i in range(_CHUNKS)
    ]
    out_cp = [
        pltpu.make_async_copy(
            o_vmem.at[pl.ds(i * _ROWS, _ROWS), :],
            o_hbm.at[pl.ds(i * _ROWS, _ROWS), :],
            sem_out.at[i],
        )
        for i in range(_CHUNKS)
    ]
    for i in range(_CHUNKS):
        in_cp[i].start()
    w = w_ref[...]
    b = b_ref[...]
    for i in range(_CHUNKS):
        in_cp[i].wait()
        sl = pl.ds(i * _ROWS, _ROWS)
        o_vmem[sl, :] = (
            jnp.dot(x_vmem[sl, :], w, preferred_element_type=jnp.float32) + b
        )
        out_cp[i].start()
    for i in range(_CHUNKS):
        out_cp[i].wait()


def kernel(feat, server_model, len_seq, mask, manfeat, emb1_w, emb2_w, emb3_w,
           emb4_w, emb5_w, k_w, o_w, cls_w, cls_b, wide_w, wide_b):
    b, k = manfeat.shape
    n = wide_w.shape[1]
    return pl.pallas_call(
        _wide_kernel,
        in_specs=[
            pl.BlockSpec(memory_space=pl.ANY),
            pl.BlockSpec(memory_space=pltpu.VMEM),
            pl.BlockSpec(memory_space=pltpu.VMEM),
        ],
        out_specs=pl.BlockSpec(memory_space=pl.ANY),
        out_shape=jax.ShapeDtypeStruct((b, n), jnp.float32),
        scratch_shapes=[
            pltpu.VMEM((b, k), jnp.float32),
            pltpu.VMEM((b, n), jnp.float32),
            pltpu.SemaphoreType.DMA((_CHUNKS,)),
            pltpu.SemaphoreType.DMA((_CHUNKS,)),
        ],
    )(manfeat, wide_w, wide_b)


# speedup vs baseline: 6.1266x; 5.3664x over previous
import jax
import jax.numpy as jnp
from jax.experimental import pallas as pl


def _wide_kernel(w_ref, x_ref, b_ref, o_ref):
    o_ref[...] = (
        jnp.dot(w_ref[...], x_ref[...], preferred_element_type=jnp.float32)
        + b_ref[...][:, None]
    )


def kernel(feat, server_model, len_seq, mask, manfeat, emb1_w, emb2_w, emb3_w,
           emb4_w, emb5_w, k_w, o_w, cls_w, cls_b, wide_w, wide_b):
    b, k = manfeat.shape
    n = wide_w.shape[1]
    xt = manfeat.T
    wt = wide_w.T
    out_t = pl.pallas_call(
        _wide_kernel,
        out_shape=jax.ShapeDtypeStruct((n, b), jnp.float32),
    )(wt, xt, wide_b)
    return out_t.T
